# baseline (speedup 1.0000x reference)
import functools

import jax
import jax.numpy as jnp
from jax import lax
from jax.experimental import pallas as pl
from jax.experimental.pallas import tpu as pltpu
from jax.experimental.pallas import tpu_sc as plsc

N = 10000
E = 320000
D_IN = 128
D_HID = 256
N_GPS = 64
N_OUT = 128

NPAD = 10240
NW = 32
RT = NPAD // 16
CKG = 120
CHG = 84
EPADG = NW * CHG * CKG

_mesh = plsc.VectorSubcoreMesh(core_axis_name="c", subcore_axis_name="s")


@functools.partial(
    pl.kernel,
    out_type=jax.ShapeDtypeStruct((2, NPAD), jnp.float32),
    mesh=_mesh,
    scratch_types=[
        pltpu.VMEM((CHG, CKG), jnp.int32),
        pltpu.VMEM((CKG,), jnp.float32),
        pltpu.VMEM((RT,), jnp.float32),
        pltpu.VMEM_SHARED((NPAD,), jnp.float32),
    ],
)
def _deg_kernel(dst_hbm, out_hbm, didx, ones, zbuf, acc):
    c = lax.axis_index("c")
    s = lax.axis_index("s")
    w = s * 2 + c
    z16 = jnp.zeros((16,), jnp.float32)
    o16 = jnp.full((16,), 1.0, jnp.float32)
    for j in range(RT // 16):
        zbuf[pl.ds(j * 16, 16)] = z16
    for j in range(CKG // 16):
        ones[pl.ds(j * 16, 16)] = o16
    pltpu.sync_copy(zbuf, acc.at[pl.ds(s * RT, RT)])
    plsc.subcore_barrier()
    pltpu.sync_copy(dst_hbm.at[w], didx)

    def body(j, carry):
        pltpu.sync_copy(ones, acc.at[didx.at[j]], add=True)
        return carry

    lax.fori_loop(0, CHG, body, 0)
    plsc.subcore_barrier()
    pltpu.sync_copy(acc.at[pl.ds(s * RT, RT)], zbuf)
    pltpu.sync_copy(zbuf, out_hbm.at[c, pl.ds(s * RT, RT)])


@functools.partial(
    pl.kernel,
    out_type=jax.ShapeDtypeStruct((2, NPAD, 128), jnp.float32),
    mesh=_mesh,
    scratch_types=[
        pltpu.VMEM((3, CKG), jnp.int32),
        pltpu.VMEM((3, CKG), jnp.int32),
        pltpu.VMEM((CKG, 128), jnp.float32),
        pltpu.VMEM((CKG, 128), jnp.float32),
        pltpu.VMEM((CKG, 128), jnp.float32),
        pltpu.VMEM_SHARED((NPAD, 128), jnp.float32),
        pltpu.SemaphoreType.DMA,
        pltpu.SemaphoreType.DMA,
        pltpu.SemaphoreType.DMA,
        pltpu.SemaphoreType.DMA,
        pltpu.SemaphoreType.DMA,
        pltpu.SemaphoreType.DMA,
        pltpu.SemaphoreType.DMA,
        pltpu.SemaphoreType.DMA,
        pltpu.SemaphoreType.DMA,
    ],
)
def _agg_kernel(tab_hbm, src_hbm, dst_hbm, out_hbm, sidx, didx,
                rows0, rows1, rows2, acc, sg0, sg1, sg2, ss0, ss1, ss2,
                sx0, sx1, sx2):
    c = lax.axis_index("c")
    s = lax.axis_index("s")
    w = s * 2 + c
    rows = (rows0, rows1, rows2)
    sg = (sg0, sg1, sg2)
    ss = (ss0, ss1, ss2)
    sx = (sx0, sx1, sx2)
    z16 = jnp.zeros((16,), jnp.float32)

    def zrow(i, carry):
        for j in range(8):
            rows0[i, pl.ds(j * 16, 16)] = z16
        return carry

    lax.fori_loop(0, 80, zrow, 0)

    def zacc(i, carry):
        pltpu.sync_copy(rows0.at[pl.ds(0, 80)],
                        acc.at[pl.ds(s * RT + i * 80, 80)])
        return carry

    lax.fori_loop(0, RT // 80, zacc, 0)
    plsc.subcore_barrier()

    pltpu.sync_copy(src_hbm.at[w, 0], sidx.at[0])
    pltpu.sync_copy(dst_hbm.at[w, 0], didx.at[0])
    pltpu.sync_copy(src_hbm.at[w, 1], sidx.at[1])
    pltpu.sync_copy(dst_hbm.at[w, 1], didx.at[1])
    pltpu.async_copy(tab_hbm.at[sidx.at[0]], rows0, sg0)
    pltpu.async_copy(tab_hbm.at[sidx.at[1]], rows1, sg1)

    nsteps = CHG // 3

    def body(i, carry):
        for b in range(3):
            j = 3 * i + b
            p = (b + 2) % 3
            more = (i > 0) if b == 0 else True
            free = True if b == 0 else (i < nsteps - 1)

            @pl.when(more)
            def _():
                pltpu.make_async_copy(
                    rows[p], acc.at[didx.at[p]], ss[p]).wait()

            @pl.when(free)
            def _():
                pltpu.async_copy(src_hbm.at[w, j + 2], sidx.at[p], sx[p])
                pltpu.async_copy(dst_hbm.at[w, j + 2], didx.at[p], sx[p])

            pltpu.make_async_copy(tab_hbm.at[sidx.at[b]], rows[b],
                                  sg[b]).wait()

            @pl.when(free)
            def _():
                pltpu.make_async_copy(
                    src_hbm.at[w, j + 2], sidx.at[p], sx[p]).wait()
                pltpu.make_async_copy(
                    dst_hbm.at[w, j + 2], didx.at[p], sx[p]).wait()
                pltpu.async_copy(tab_hbm.at[sidx.at[p]], rows[p], sg[p])

            pltpu.async_copy(rows[b], acc.at[didx.at[b]], ss[b], add=True)
        return carry

    lax.fori_loop(0, nsteps, body, 0)
    lastp = (CHG - 1) % 3
    pltpu.make_async_copy(rows[lastp], acc.at[didx.at[lastp]],
                          ss[lastp]).wait()
    plsc.subcore_barrier()

    def wb(i, carry):
        pltpu.sync_copy(acc.at[pl.ds(s * RT + i * 80, 80)],
                        rows0.at[pl.ds(0, 80)])
        pltpu.sync_copy(rows0.at[pl.ds(0, 80)],
                        out_hbm.at[c, pl.ds(s * RT + i * 80, 80)])
        return carry

    lax.fori_loop(0, RT // 80, wb, 0)


_PR = CKG // 8


@functools.partial(
    pl.kernel,
    out_type=jax.ShapeDtypeStruct((NW, CHG, _PR, 128), jnp.float32),
    mesh=_mesh,
    scratch_types=[
        pltpu.VMEM((CHG, CKG), jnp.int32),
        pltpu.VMEM((CHG, CKG), jnp.int32),
        pltpu.VMEM((CKG, 128), jnp.float32),
        pltpu.VMEM((CKG, 128), jnp.float32),
        pltpu.VMEM((CKG, 128), jnp.float32),
        pltpu.VMEM((CKG, 128), jnp.float32),
        pltpu.VMEM((CKG, 128), jnp.float32),
        pltpu.VMEM((CKG, 128), jnp.float32),
        pltpu.VMEM((_PR, 128), jnp.float32),
        pltpu.VMEM((_PR, 128), jnp.float32),
        pltpu.VMEM((_PR, 128), jnp.float32),
        pltpu.SemaphoreType.DMA,
        pltpu.SemaphoreType.DMA,
        pltpu.SemaphoreType.DMA,
        pltpu.SemaphoreType.DMA,
        pltpu.SemaphoreType.DMA,
        pltpu.SemaphoreType.DMA,
    ],
)
def _edge_kernel(zn_hbm, src_hbm, dst_hbm, out_hbm, sidx, didx,
                 zs0, zd0, zs1, zd1, zs2, zd2, pb0, pb1, pb2,
                 sg0, sg1, sg2, so0, so1, so2):
    c = lax.axis_index("c")
    s = lax.axis_index("s")
    w = s * 2 + c
    zs = (zs0, zs1, zs2)
    zd = (zd0, zd1, zd2)
    pb = (pb0, pb1, pb2)
    sg = (sg0, sg1, sg2)
    so = (so0, so1, so2)
    pltpu.sync_copy(src_hbm.at[w], sidx)
    pltpu.sync_copy(dst_hbm.at[w], didx)

    def _compute(zsb, zdb, pbuf):
        @plsc.parallel_loop(0, _PR, unroll=3)
        def _edot(r):
            for k in range(8):
                e = r * 8 + k
                p = zsb[e, pl.ds(0, 16)] * zdb[e, pl.ds(0, 16)]
                for b in range(1, 4):
                    p = p + zsb[e, pl.ds(b * 16, 16)] * zdb[e, pl.ds(b * 16, 16)]
                pbuf[r, pl.ds(k * 16, 16)] = p

    pltpu.async_copy(zn_hbm.at[sidx.at[0]], zs0, sg0)
    pltpu.async_copy(zn_hbm.at[didx.at[0]], zd0, sg0)
    pltpu.async_copy(zn_hbm.at[sidx.at[1]], zs1, sg1)
    pltpu.async_copy(zn_hbm.at[didx.at[1]], zd1, sg1)
    nsteps = CHG // 3

    def body(i, carry):
        for b in range(3):
            j = 3 * i + b
            p = (b + 2) % 3
            pltpu.make_async_copy(zn_hbm.at[sidx.at[j]], zs[b], sg[b]).wait()
            pltpu.make_async_copy(zn_hbm.at[didx.at[j]], zd[b], sg[b]).wait()

            free = True if b == 0 else (i < nsteps - 1)

            @pl.when(free)
            def _():
                pltpu.async_copy(zn_hbm.at[sidx.at[j + 2]], zs[p], sg[p])
                pltpu.async_copy(zn_hbm.at[didx.at[j + 2]], zd[p], sg[p])

            @pl.when(i > 0)
            def _():
                pltpu.make_async_copy(pb[b], out_hbm.at[w, j], so[b]).wait()

            _compute(zs[b], zd[b], pb[b])
            pltpu.async_copy(pb[b], out_hbm.at[w, j], so[b])
        return carry

    lax.fori_loop(0, nsteps, body, 0)
    for b in range(3):
        pltpu.make_async_copy(pb[b], out_hbm.at[w, CHG - 3 + b],
                              so[b]).wait()


_BR = 2048


def _tc1_body(degp_ref, x_ref, t1_ref, dinv_ref):
    deg = degp_ref[:, 0:1] + degp_ref[:, 1:2] + 1.0
    dinv = lax.rsqrt(jnp.maximum(deg, 1.0))
    t1_ref[...] = jnp.log1p(x_ref[...]) * dinv
    dinv_ref[...] = dinv


def _tc1(degp_t, x_pad):
    return pl.pallas_call(
        _tc1_body,
        grid=(NPAD // _BR,),
        in_specs=[
            pl.BlockSpec((_BR, 2), lambda i: (i, 0)),
            pl.BlockSpec((_BR, D_IN), lambda i: (i, 0)),
        ],
        out_specs=[
            pl.BlockSpec((_BR, D_IN), lambda i: (i, 0)),
            pl.BlockSpec((_BR, 1), lambda i: (i, 0)),
        ],
        out_shape=[
            jax.ShapeDtypeStruct((NPAD, D_IN), jnp.float32),
            jax.ShapeDtypeStruct((NPAD, 1), jnp.float32),
        ],
    )(degp_t, x_pad)


def _tc2_body(p_ref, t1_ref, dinv_ref, w1_ref, wmu_ref, wls_ref,
              t2_ref):
    dv = dinv_ref[...]
    agg1 = (p_ref[0] + p_ref[1] + t1_ref[...]) * dv
    h = jnp.maximum(
        jnp.dot(agg1, w1_ref[...], preferred_element_type=jnp.float32), 0.0)
    hm = jnp.dot(h, wmu_ref[...], preferred_element_type=jnp.float32)
    hs = jnp.dot(h, wls_ref[...], preferred_element_type=jnp.float32)
    t2_ref[...] = jnp.concatenate([hm, hs], axis=1) * dv


def _tc2(parts, t1, dinv, W1, W_mu, W_logstd):
    return pl.pallas_call(
        _tc2_body,
        grid=(NPAD // _BR,),
        in_specs=[
            pl.BlockSpec((2, _BR, D_IN), lambda i: (0, i, 0)),
            pl.BlockSpec((_BR, D_IN), lambda i: (i, 0)),
            pl.BlockSpec((_BR, 1), lambda i: (i, 0)),
            pl.BlockSpec((D_IN, D_HID), lambda i: (0, 0)),
            pl.BlockSpec((D_HID, N_GPS), lambda i: (0, 0)),
            pl.BlockSpec((D_HID, N_GPS), lambda i: (0, 0)),
        ],
        out_specs=pl.BlockSpec((_BR, 2 * N_GPS), lambda i: (i, 0)),
        out_shape=jax.ShapeDtypeStruct((NPAD, 2 * N_GPS), jnp.float32),
    )(parts, t1, dinv, W1, W_mu, W_logstd)


def _tc3_body(q_ref, t2_ref, dinv_ref, wge_ref, mask_ref,
              mu_ref, ls_ref, zn_ref, gep_ref):
    dv = dinv_ref[...]
    m = (q_ref[0] + q_ref[1] + t2_ref[...]) * dv
    mu = m[:, :N_GPS]
    ls = m[:, N_GPS:]
    nrm = jnp.sqrt(jnp.sum(mu * mu, axis=1, keepdims=True))
    zn = mu / (nrm + 1e-8)
    wm = wge_ref[...] * mask_ref[...]
    gl = jnp.dot(mu, wm, preferred_element_type=jnp.float32)
    gmax = jnp.max(gl, axis=1, keepdims=True)
    ge = jnp.exp(gl - gmax)
    gep = ge / jnp.sum(ge, axis=1, keepdims=True)
    mu_ref[...] = mu
    ls_ref[...] = ls
    zn_ref[...] = jnp.concatenate([zn, jnp.zeros_like(zn)], axis=1)
    gep_ref[...] = gep


def _tc3(parts, t2, dinv, W_ge, mask):
    return pl.pallas_call(
        _tc3_body,
        grid=(NPAD // _BR,),
        in_specs=[
            pl.BlockSpec((2, _BR, 2 * N_GPS), lambda i: (0, i, 0)),
            pl.BlockSpec((_BR, 2 * N_GPS), lambda i: (i, 0)),
            pl.BlockSpec((_BR, 1), lambda i: (i, 0)),
            pl.BlockSpec((N_GPS, N_OUT), lambda i: (0, 0)),
            pl.BlockSpec((N_GPS, N_OUT), lambda i: (0, 0)),
        ],
        out_specs=[
            pl.BlockSpec((_BR, N_GPS), lambda i: (i, 0)),
            pl.BlockSpec((_BR, N_GPS), lambda i: (i, 0)),
            pl.BlockSpec((_BR, 2 * N_GPS), lambda i: (i, 0)),
            pl.BlockSpec((_BR, N_OUT), lambda i: (i, 0)),
        ],
        out_shape=[
            jax.ShapeDtypeStruct((NPAD, N_GPS), jnp.float32),
            jax.ShapeDtypeStruct((NPAD, N_GPS), jnp.float32),
            jax.ShapeDtypeStruct((NPAD, 2 * N_GPS), jnp.float32),
            jax.ShapeDtypeStruct((NPAD, N_OUT), jnp.float32),
        ],
    )(parts, t2, dinv, W_ge, mask)


_R16 = NW * CHG * _PR
_BRE = 8064


def _tc4_body(p_ref, out_ref):
    p = p_ref[...]
    cols = [jnp.sum(p[:, k * 16:(k + 1) * 16], axis=1) for k in range(8)]
    out_ref[...] = jnp.stack(cols, axis=0)


def _tc4(pfold):
    return pl.pallas_call(
        _tc4_body,
        grid=(_R16 // _BRE,),
        in_specs=[pl.BlockSpec((_BRE, 128), lambda i: (i, 0))],
        out_specs=pl.BlockSpec((8, _BRE), lambda i: (0, i)),
        out_shape=jax.ShapeDtypeStruct((8, _R16), jnp.float32),
    )(pfold)


def kernel(x, edge_index, W1, W_mu, W_logstd, W_ge, mask):
    src = edge_index[0]
    dst = edge_index[1]
    padg = (N + jnp.arange(EPADG - E, dtype=jnp.int32) % (NPAD - N)).astype(
        jnp.int32)
    srcg = jnp.concatenate([src, padg]).reshape(NW, CHG, CKG)
    dstg = jnp.concatenate([dst, padg]).reshape(NW, CHG, CKG)
    x_pad = jnp.pad(x, ((0, NPAD - N), (0, 0)))

    deg_parts = _deg_kernel(dstg)
    t1, dinv = _tc1(deg_parts.T, x_pad)
    parts1 = _agg_kernel(t1, srcg, dstg)
    t2 = _tc2(parts1, t1, dinv, W1, W_mu, W_logstd)
    parts2 = _agg_kernel(t2, srcg, dstg)
    mu_p, ls_p, zn_p, gep_p = _tc3(parts2, t2, dinv, W_ge, mask)
    pfold = _edge_kernel(zn_p, srcg, dstg).reshape(_R16, 128)
    out4 = _tc4(pfold)
    elog = (out4.reshape(8, NW * CHG, _PR)
            .transpose(1, 2, 0).reshape(-1)[:E])
    return (elog, gep_p[:N], mu_p[:N], ls_p[:N])


# speedup vs baseline: 1.0020x; 1.0020x over previous
import functools

import jax
import jax.numpy as jnp
from jax import lax
from jax.experimental import pallas as pl
from jax.experimental.pallas import tpu as pltpu
from jax.experimental.pallas import tpu_sc as plsc

N = 10000
E = 320000
D_IN = 128
D_HID = 256
N_GPS = 64
N_OUT = 128

NPAD = 10240
NW = 32
RT = NPAD // 16
CKG = 120
CHG = 84
EPADG = NW * CHG * CKG
CHD = 80
EPADD = NW * CHD * 128

_mesh = plsc.VectorSubcoreMesh(core_axis_name="c", subcore_axis_name="s")


@functools.partial(
    pl.kernel,
    out_type=jax.ShapeDtypeStruct((2, NPAD), jnp.float32),
    mesh=_mesh,
    scratch_types=[
        pltpu.VMEM((CHD, 128), jnp.int32),
        pltpu.VMEM((128,), jnp.float32),
        pltpu.VMEM((RT,), jnp.float32),
        pltpu.VMEM_SHARED((NPAD,), jnp.float32),
    ],
)
def _deg_kernel(dst_hbm, out_hbm, didx, ones, zbuf, acc):
    c = lax.axis_index("c")
    s = lax.axis_index("s")
    w = s * 2 + c
    z16 = jnp.zeros((16,), jnp.float32)
    o16 = jnp.full((16,), 1.0, jnp.float32)
    for j in range(RT // 16):
        zbuf[pl.ds(j * 16, 16)] = z16
    for j in range(8):
        ones[pl.ds(j * 16, 16)] = o16
    pltpu.sync_copy(zbuf, acc.at[pl.ds(s * RT, RT)])
    plsc.subcore_barrier()
    pltpu.sync_copy(dst_hbm.at[w], didx)

    def body(j, carry):
        pltpu.sync_copy(ones, acc.at[didx.at[j]], add=True)
        return carry

    lax.fori_loop(0, CHD, body, 0)
    plsc.subcore_barrier()
    pltpu.sync_copy(acc.at[pl.ds(s * RT, RT)], zbuf)
    pltpu.sync_copy(zbuf, out_hbm.at[c, pl.ds(s * RT, RT)])


@functools.partial(
    pl.kernel,
    out_type=jax.ShapeDtypeStruct((2, NPAD, 128), jnp.float32),
    mesh=_mesh,
    scratch_types=[
        pltpu.VMEM((3, CKG), jnp.int32),
        pltpu.VMEM((3, CKG), jnp.int32),
        pltpu.VMEM((CKG, 128), jnp.float32),
        pltpu.VMEM((CKG, 128), jnp.float32),
        pltpu.VMEM((CKG, 128), jnp.float32),
        pltpu.VMEM_SHARED((NPAD, 128), jnp.float32),
        pltpu.SemaphoreType.DMA,
        pltpu.SemaphoreType.DMA,
        pltpu.SemaphoreType.DMA,
        pltpu.SemaphoreType.DMA,
        pltpu.SemaphoreType.DMA,
        pltpu.SemaphoreType.DMA,
        pltpu.SemaphoreType.DMA,
        pltpu.SemaphoreType.DMA,
        pltpu.SemaphoreType.DMA,
    ],
)
def _agg_kernel(tab_hbm, src_hbm, dst_hbm, out_hbm, sidx, didx,
                rows0, rows1, rows2, acc, sg0, sg1, sg2, ss0, ss1, ss2,
                sx0, sx1, sx2):
    c = lax.axis_index("c")
    s = lax.axis_index("s")
    w = s * 2 + c
    rows = (rows0, rows1, rows2)
    sg = (sg0, sg1, sg2)
    ss = (ss0, ss1, ss2)
    sx = (sx0, sx1, sx2)
    z16 = jnp.zeros((16,), jnp.float32)

    def zrow(i, carry):
        for j in range(8):
            rows0[i, pl.ds(j * 16, 16)] = z16
        return carry

    lax.fori_loop(0, 80, zrow, 0)

    def zacc(i, carry):
        pltpu.sync_copy(rows0.at[pl.ds(0, 80)],
                        acc.at[pl.ds(s * RT + i * 80, 80)])
        return carry

    lax.fori_loop(0, RT // 80, zacc, 0)
    plsc.subcore_barrier()

    pltpu.sync_copy(src_hbm.at[w, 0], sidx.at[0])
    pltpu.sync_copy(dst_hbm.at[w, 0], didx.at[0])
    pltpu.sync_copy(src_hbm.at[w, 1], sidx.at[1])
    pltpu.sync_copy(dst_hbm.at[w, 1], didx.at[1])
    pltpu.async_copy(tab_hbm.at[sidx.at[0]], rows0, sg0)
    pltpu.async_copy(tab_hbm.at[sidx.at[1]], rows1, sg1)

    nsteps = CHG // 3

    def body(i, carry):
        for b in range(3):
            j = 3 * i + b
            p = (b + 2) % 3
            more = (i > 0) if b == 0 else True
            free = True if b == 0 else (i < nsteps - 1)

            @pl.when(more)
            def _():
                pltpu.make_async_copy(
                    rows[p], acc.at[didx.at[p]], ss[p]).wait()

            @pl.when(free)
            def _():
                pltpu.async_copy(src_hbm.at[w, j + 2], sidx.at[p], sx[p])
                pltpu.async_copy(dst_hbm.at[w, j + 2], didx.at[p], sx[p])

            pltpu.make_async_copy(tab_hbm.at[sidx.at[b]], rows[b],
                                  sg[b]).wait()

            @pl.when(free)
            def _():
                pltpu.make_async_copy(
                    src_hbm.at[w, j + 2], sidx.at[p], sx[p]).wait()
                pltpu.make_async_copy(
                    dst_hbm.at[w, j + 2], didx.at[p], sx[p]).wait()
                pltpu.async_copy(tab_hbm.at[sidx.at[p]], rows[p], sg[p])

            pltpu.async_copy(rows[b], acc.at[didx.at[b]], ss[b], add=True)
        return carry

    lax.fori_loop(0, nsteps, body, 0)
    lastp = (CHG - 1) % 3
    pltpu.make_async_copy(rows[lastp], acc.at[didx.at[lastp]],
                          ss[lastp]).wait()
    plsc.subcore_barrier()

    def wb(i, carry):
        pltpu.sync_copy(acc.at[pl.ds(s * RT + i * 80, 80)],
                        rows0.at[pl.ds(0, 80)])
        pltpu.sync_copy(rows0.at[pl.ds(0, 80)],
                        out_hbm.at[c, pl.ds(s * RT + i * 80, 80)])
        return carry

    lax.fori_loop(0, RT // 80, wb, 0)


_PR = CKG // 8


@functools.partial(
    pl.kernel,
    out_type=jax.ShapeDtypeStruct((NW, CHG, _PR, 128), jnp.float32),
    mesh=_mesh,
    scratch_types=[
        pltpu.VMEM((CHG, CKG), jnp.int32),
        pltpu.VMEM((CHG, CKG), jnp.int32),
        pltpu.VMEM((CKG, 128), jnp.float32),
        pltpu.VMEM((CKG, 128), jnp.float32),
        pltpu.VMEM((CKG, 128), jnp.float32),
        pltpu.VMEM((CKG, 128), jnp.float32),
        pltpu.VMEM((CKG, 128), jnp.float32),
        pltpu.VMEM((CKG, 128), jnp.float32),
        pltpu.VMEM((_PR, 128), jnp.float32),
        pltpu.VMEM((_PR, 128), jnp.float32),
        pltpu.VMEM((_PR, 128), jnp.float32),
        pltpu.SemaphoreType.DMA,
        pltpu.SemaphoreType.DMA,
        pltpu.SemaphoreType.DMA,
        pltpu.SemaphoreType.DMA,
        pltpu.SemaphoreType.DMA,
        pltpu.SemaphoreType.DMA,
    ],
)
def _edge_kernel(zn_hbm, src_hbm, dst_hbm, out_hbm, sidx, didx,
                 zs0, zd0, zs1, zd1, zs2, zd2, pb0, pb1, pb2,
                 sg0, sg1, sg2, so0, so1, so2):
    c = lax.axis_index("c")
    s = lax.axis_index("s")
    w = s * 2 + c
    zs = (zs0, zs1, zs2)
    zd = (zd0, zd1, zd2)
    pb = (pb0, pb1, pb2)
    sg = (sg0, sg1, sg2)
    so = (so0, so1, so2)
    pltpu.sync_copy(src_hbm.at[w], sidx)
    pltpu.sync_copy(dst_hbm.at[w], didx)

    def _compute(zsb, zdb, pbuf):
        @plsc.parallel_loop(0, _PR, unroll=3)
        def _edot(r):
            for k in range(8):
                e = r * 8 + k
                p = zsb[e, pl.ds(0, 16)] * zdb[e, pl.ds(0, 16)]
                for b in range(1, 4):
                    p = p + zsb[e, pl.ds(b * 16, 16)] * zdb[e, pl.ds(b * 16, 16)]
                pbuf[r, pl.ds(k * 16, 16)] = p

    pltpu.async_copy(zn_hbm.at[sidx.at[0]], zs0, sg0)
    pltpu.async_copy(zn_hbm.at[didx.at[0]], zd0, sg0)
    pltpu.async_copy(zn_hbm.at[sidx.at[1]], zs1, sg1)
    pltpu.async_copy(zn_hbm.at[didx.at[1]], zd1, sg1)
    nsteps = CHG // 3

    def body(i, carry):
        for b in range(3):
            j = 3 * i + b
            p = (b + 2) % 3
            pltpu.make_async_copy(zn_hbm.at[sidx.at[j]], zs[b], sg[b]).wait()
            pltpu.make_async_copy(zn_hbm.at[didx.at[j]], zd[b], sg[b]).wait()

            free = True if b == 0 else (i < nsteps - 1)

            @pl.when(free)
            def _():
                pltpu.async_copy(zn_hbm.at[sidx.at[j + 2]], zs[p], sg[p])
                pltpu.async_copy(zn_hbm.at[didx.at[j + 2]], zd[p], sg[p])

            @pl.when(i > 0)
            def _():
                pltpu.make_async_copy(pb[b], out_hbm.at[w, j], so[b]).wait()

            _compute(zs[b], zd[b], pb[b])
            pltpu.async_copy(pb[b], out_hbm.at[w, j], so[b])
        return carry

    lax.fori_loop(0, nsteps, body, 0)
    for b in range(3):
        pltpu.make_async_copy(pb[b], out_hbm.at[w, CHG - 3 + b],
                              so[b]).wait()


_BR = 2048


def _tc1_body(degp_ref, x_ref, t1_ref, dinv_ref):
    deg = degp_ref[:, 0:1] + degp_ref[:, 1:2] + 1.0
    dinv = lax.rsqrt(jnp.maximum(deg, 1.0))
    t1_ref[...] = jnp.log1p(x_ref[...]) * dinv
    dinv_ref[...] = dinv


def _tc1(degp_t, x_pad):
    return pl.pallas_call(
        _tc1_body,
        grid=(NPAD // _BR,),
        in_specs=[
            pl.BlockSpec((_BR, 2), lambda i: (i, 0)),
            pl.BlockSpec((_BR, D_IN), lambda i: (i, 0)),
        ],
        out_specs=[
            pl.BlockSpec((_BR, D_IN), lambda i: (i, 0)),
            pl.BlockSpec((_BR, 1), lambda i: (i, 0)),
        ],
        out_shape=[
            jax.ShapeDtypeStruct((NPAD, D_IN), jnp.float32),
            jax.ShapeDtypeStruct((NPAD, 1), jnp.float32),
        ],
    )(degp_t, x_pad)


def _tc2_body(p_ref, t1_ref, dinv_ref, w1_ref, wmu_ref, wls_ref,
              t2_ref):
    dv = dinv_ref[...]
    agg1 = (p_ref[0] + p_ref[1] + t1_ref[...]) * dv
    h = jnp.maximum(
        jnp.dot(agg1, w1_ref[...], preferred_element_type=jnp.float32), 0.0)
    hm = jnp.dot(h, wmu_ref[...], preferred_element_type=jnp.float32)
    hs = jnp.dot(h, wls_ref[...], preferred_element_type=jnp.float32)
    t2_ref[...] = jnp.concatenate([hm, hs], axis=1) * dv


def _tc2(parts, t1, dinv, W1, W_mu, W_logstd):
    return pl.pallas_call(
        _tc2_body,
        grid=(NPAD // _BR,),
        in_specs=[
            pl.BlockSpec((2, _BR, D_IN), lambda i: (0, i, 0)),
            pl.BlockSpec((_BR, D_IN), lambda i: (i, 0)),
            pl.BlockSpec((_BR, 1), lambda i: (i, 0)),
            pl.BlockSpec((D_IN, D_HID), lambda i: (0, 0)),
            pl.BlockSpec((D_HID, N_GPS), lambda i: (0, 0)),
            pl.BlockSpec((D_HID, N_GPS), lambda i: (0, 0)),
        ],
        out_specs=pl.BlockSpec((_BR, 2 * N_GPS), lambda i: (i, 0)),
        out_shape=jax.ShapeDtypeStruct((NPAD, 2 * N_GPS), jnp.float32),
    )(parts, t1, dinv, W1, W_mu, W_logstd)


def _tc3_body(q_ref, t2_ref, dinv_ref, wge_ref, mask_ref,
              mu_ref, ls_ref, zn_ref, gep_ref):
    dv = dinv_ref[...]
    m = (q_ref[0] + q_ref[1] + t2_ref[...]) * dv
    mu = m[:, :N_GPS]
    ls = m[:, N_GPS:]
    nrm = jnp.sqrt(jnp.sum(mu * mu, axis=1, keepdims=True))
    zn = mu / (nrm + 1e-8)
    wm = wge_ref[...] * mask_ref[...]
    gl = jnp.dot(mu, wm, preferred_element_type=jnp.float32)
    gmax = jnp.max(gl, axis=1, keepdims=True)
    ge = jnp.exp(gl - gmax)
    gep = ge / jnp.sum(ge, axis=1, keepdims=True)
    mu_ref[...] = mu
    ls_ref[...] = ls
    zn_ref[...] = jnp.concatenate([zn, jnp.zeros_like(zn)], axis=1)
    gep_ref[...] = gep


def _tc3(parts, t2, dinv, W_ge, mask):
    return pl.pallas_call(
        _tc3_body,
        grid=(NPAD // _BR,),
        in_specs=[
            pl.BlockSpec((2, _BR, 2 * N_GPS), lambda i: (0, i, 0)),
            pl.BlockSpec((_BR, 2 * N_GPS), lambda i: (i, 0)),
            pl.BlockSpec((_BR, 1), lambda i: (i, 0)),
            pl.BlockSpec((N_GPS, N_OUT), lambda i: (0, 0)),
            pl.BlockSpec((N_GPS, N_OUT), lambda i: (0, 0)),
        ],
        out_specs=[
            pl.BlockSpec((_BR, N_GPS), lambda i: (i, 0)),
            pl.BlockSpec((_BR, N_GPS), lambda i: (i, 0)),
            pl.BlockSpec((_BR, 2 * N_GPS), lambda i: (i, 0)),
            pl.BlockSpec((_BR, N_OUT), lambda i: (i, 0)),
        ],
        out_shape=[
            jax.ShapeDtypeStruct((NPAD, N_GPS), jnp.float32),
            jax.ShapeDtypeStruct((NPAD, N_GPS), jnp.float32),
            jax.ShapeDtypeStruct((NPAD, 2 * N_GPS), jnp.float32),
            jax.ShapeDtypeStruct((NPAD, N_OUT), jnp.float32),
        ],
    )(parts, t2, dinv, W_ge, mask)


_R16 = NW * CHG * _PR
_BRE = 8064


def _tc4_body(p_ref, out_ref):
    p = p_ref[...]
    cols = [jnp.sum(p[:, k * 16:(k + 1) * 16], axis=1) for k in range(8)]
    out_ref[...] = jnp.stack(cols, axis=0)


def _tc4(pfold):
    return pl.pallas_call(
        _tc4_body,
        grid=(_R16 // _BRE,),
        in_specs=[pl.BlockSpec((_BRE, 128), lambda i: (i, 0))],
        out_specs=pl.BlockSpec((8, _BRE), lambda i: (0, i)),
        out_shape=jax.ShapeDtypeStruct((8, _R16), jnp.float32),
    )(pfold)


def kernel(x, edge_index, W1, W_mu, W_logstd, W_ge, mask):
    src = edge_index[0]
    dst = edge_index[1]
    padg = (N + jnp.arange(EPADG - E, dtype=jnp.int32) % (NPAD - N)).astype(
        jnp.int32)
    srcg = jnp.concatenate([src, padg]).reshape(NW, CHG, CKG)
    dstg = jnp.concatenate([dst, padg]).reshape(NW, CHG, CKG)
    padd = (N + jnp.arange(EPADD - E, dtype=jnp.int32) % (NPAD - N)).astype(
        jnp.int32)
    dstd = jnp.concatenate([dst, padd]).reshape(NW, CHD, 128)
    x_pad = jnp.pad(x, ((0, NPAD - N), (0, 0)))

    deg_parts = _deg_kernel(dstd)
    t1, dinv = _tc1(deg_parts.T, x_pad)
    parts1 = _agg_kernel(t1, srcg, dstg)
    t2 = _tc2(parts1, t1, dinv, W1, W_mu, W_logstd)
    parts2 = _agg_kernel(t2, srcg, dstg)
    mu_p, ls_p, zn_p, gep_p = _tc3(parts2, t2, dinv, W_ge, mask)
    pfold = _edge_kernel(zn_p, srcg, dstg).reshape(_R16, 128)
    out4 = _tc4(pfold)
    elog = (out4.reshape(8, NW * CHG, _PR)
            .transpose(1, 2, 0).reshape(-1)[:E])
    return (elog, gep_p[:N], mu_p[:N], ls_p[:N])


# speedup vs baseline: 1.0152x; 1.0131x over previous
import functools

import jax
import jax.numpy as jnp
from jax import lax
from jax.experimental import pallas as pl
from jax.experimental.pallas import tpu as pltpu
from jax.experimental.pallas import tpu_sc as plsc

N = 10000
E = 320000
D_IN = 128
D_HID = 256
N_GPS = 64
N_OUT = 128

NPAD = 10240
NW = 32
RT = NPAD // 16
CKG = 120
CHG = 84
EPADG = NW * CHG * CKG
CHD = 80
EPADD = NW * CHD * 128

_mesh = plsc.VectorSubcoreMesh(core_axis_name="c", subcore_axis_name="s")


@functools.partial(
    pl.kernel,
    out_type=jax.ShapeDtypeStruct((2, NPAD), jnp.float32),
    mesh=_mesh,
    scratch_types=[
        pltpu.VMEM((CHD, 128), jnp.int32),
        pltpu.VMEM((128,), jnp.float32),
        pltpu.VMEM((RT,), jnp.float32),
        pltpu.VMEM_SHARED((NPAD,), jnp.float32),
    ],
)
def _deg_kernel(dst_hbm, out_hbm, didx, ones, zbuf, acc):
    c = lax.axis_index("c")
    s = lax.axis_index("s")
    w = s * 2 + c
    z16 = jnp.zeros((16,), jnp.float32)
    o16 = jnp.full((16,), 1.0, jnp.float32)
    for j in range(RT // 16):
        zbuf[pl.ds(j * 16, 16)] = z16
    for j in range(8):
        ones[pl.ds(j * 16, 16)] = o16
    pltpu.sync_copy(zbuf, acc.at[pl.ds(s * RT, RT)])
    plsc.subcore_barrier()
    pltpu.sync_copy(dst_hbm.at[w], didx)

    def body(j, carry):
        pltpu.sync_copy(ones, acc.at[didx.at[j]], add=True)
        return carry

    lax.fori_loop(0, CHD, body, 0)
    plsc.subcore_barrier()
    pltpu.sync_copy(acc.at[pl.ds(s * RT, RT)], zbuf)
    pltpu.sync_copy(zbuf, out_hbm.at[c, pl.ds(s * RT, RT)])


@functools.partial(
    pl.kernel,
    out_type=jax.ShapeDtypeStruct((2, NPAD, 128), jnp.float32),
    mesh=_mesh,
    scratch_types=[
        pltpu.VMEM((3, CKG), jnp.int32),
        pltpu.VMEM((3, CKG), jnp.int32),
        pltpu.VMEM((CKG, 128), jnp.float32),
        pltpu.VMEM((CKG, 128), jnp.float32),
        pltpu.VMEM((CKG, 128), jnp.float32),
        pltpu.VMEM_SHARED((NPAD, 128), jnp.float32),
        pltpu.SemaphoreType.DMA,
        pltpu.SemaphoreType.DMA,
        pltpu.SemaphoreType.DMA,
        pltpu.SemaphoreType.DMA,
        pltpu.SemaphoreType.DMA,
        pltpu.SemaphoreType.DMA,
        pltpu.SemaphoreType.DMA,
        pltpu.SemaphoreType.DMA,
        pltpu.SemaphoreType.DMA,
    ],
)
def _agg_kernel(tab_hbm, src_hbm, dst_hbm, out_hbm, sidx, didx,
                rows0, rows1, rows2, acc, sg0, sg1, sg2, ss0, ss1, ss2,
                sx0, sx1, sx2):
    c = lax.axis_index("c")
    s = lax.axis_index("s")
    w = s * 2 + c
    rows = (rows0, rows1, rows2)
    sg = (sg0, sg1, sg2)
    ss = (ss0, ss1, ss2)
    sx = (sx0, sx1, sx2)
    z16 = jnp.zeros((16,), jnp.float32)

    def zrow(i, carry):
        for j in range(8):
            rows0[i, pl.ds(j * 16, 16)] = z16
        return carry

    lax.fori_loop(0, 80, zrow, 0)

    def zacc(i, carry):
        pltpu.sync_copy(rows0.at[pl.ds(0, 80)],
                        acc.at[pl.ds(s * RT + i * 80, 80)])
        return carry

    lax.fori_loop(0, RT // 80, zacc, 0)
    plsc.subcore_barrier()

    pltpu.sync_copy(src_hbm.at[w, 0], sidx.at[0])
    pltpu.sync_copy(dst_hbm.at[w, 0], didx.at[0])
    pltpu.sync_copy(src_hbm.at[w, 1], sidx.at[1])
    pltpu.sync_copy(dst_hbm.at[w, 1], didx.at[1])
    pltpu.async_copy(tab_hbm.at[sidx.at[0]], rows0, sg0)
    pltpu.async_copy(tab_hbm.at[sidx.at[1]], rows1, sg1)

    nsteps = CHG // 3

    def body(i, carry):
        for b in range(3):
            j = 3 * i + b
            p = (b + 2) % 3
            more = (i > 0) if b == 0 else True
            free = True if b == 0 else (i < nsteps - 1)

            @pl.when(more)
            def _():
                pltpu.make_async_copy(
                    rows[p], acc.at[didx.at[p]], ss[p]).wait()

            @pl.when(free)
            def _():
                pltpu.async_copy(src_hbm.at[w, j + 2], sidx.at[p], sx[p])
                pltpu.async_copy(dst_hbm.at[w, j + 2], didx.at[p], sx[p])

            pltpu.make_async_copy(tab_hbm.at[sidx.at[b]], rows[b],
                                  sg[b]).wait()

            @pl.when(free)
            def _():
                pltpu.make_async_copy(
                    src_hbm.at[w, j + 2], sidx.at[p], sx[p]).wait()
                pltpu.make_async_copy(
                    dst_hbm.at[w, j + 2], didx.at[p], sx[p]).wait()
                pltpu.async_copy(tab_hbm.at[sidx.at[p]], rows[p], sg[p])

            pltpu.async_copy(rows[b], acc.at[didx.at[b]], ss[b], add=True)
        return carry

    lax.fori_loop(0, nsteps, body, 0)
    lastp = (CHG - 1) % 3
    pltpu.make_async_copy(rows[lastp], acc.at[didx.at[lastp]],
                          ss[lastp]).wait()
    plsc.subcore_barrier()

    def wb(i, carry):
        pltpu.sync_copy(acc.at[pl.ds(s * RT + i * 80, 80)],
                        rows0.at[pl.ds(0, 80)])
        pltpu.sync_copy(rows0.at[pl.ds(0, 80)],
                        out_hbm.at[c, pl.ds(s * RT + i * 80, 80)])
        return carry

    lax.fori_loop(0, RT // 80, wb, 0)


@functools.partial(
    pl.kernel,
    out_type=jax.ShapeDtypeStruct((NW, CHD, 16, 128), jnp.float32),
    mesh=_mesh,
    scratch_types=[
        pltpu.VMEM((CHD, 128), jnp.int32),
        pltpu.VMEM((CHD, 128), jnp.int32),
        pltpu.VMEM((128, 128), jnp.float32),
        pltpu.VMEM((128, 128), jnp.float32),
        pltpu.VMEM((128, 128), jnp.float32),
        pltpu.VMEM((128, 128), jnp.float32),
        pltpu.VMEM((16, 128), jnp.float32),
        pltpu.VMEM((16, 128), jnp.float32),
        pltpu.SemaphoreType.DMA,
        pltpu.SemaphoreType.DMA,
        pltpu.SemaphoreType.DMA,
        pltpu.SemaphoreType.DMA,
    ],
)
def _edge_kernel(zn_hbm, src_hbm, dst_hbm, out_hbm, sidx, didx,
                 zs0, zd0, zs1, zd1, pb0, pb1, sg0, sg1, so0, so1):
    c = lax.axis_index("c")
    s = lax.axis_index("s")
    w = s * 2 + c
    pltpu.sync_copy(src_hbm.at[w], sidx)
    pltpu.sync_copy(dst_hbm.at[w], didx)

    def _compute(zs, zd, pbuf):
        @plsc.parallel_loop(0, 16, unroll=4)
        def _edot(r):
            for k in range(8):
                e = r * 8 + k
                p = zs[e, pl.ds(0, 16)] * zd[e, pl.ds(0, 16)]
                for b in range(1, 4):
                    p = p + zs[e, pl.ds(b * 16, 16)] * zd[e, pl.ds(b * 16, 16)]
                pbuf[r, pl.ds(k * 16, 16)] = p

    pltpu.async_copy(zn_hbm.at[sidx.at[0]], zs0, sg0)
    pltpu.async_copy(zn_hbm.at[didx.at[0]], zd0, sg0)
    pltpu.async_copy(zn_hbm.at[sidx.at[1]], zs1, sg1)
    pltpu.async_copy(zn_hbm.at[didx.at[1]], zd1, sg1)

    def body(i, carry):
        j0 = 2 * i
        j1 = j0 + 1
        pltpu.make_async_copy(zn_hbm.at[sidx.at[j0]], zs0, sg0).wait()
        pltpu.make_async_copy(zn_hbm.at[didx.at[j0]], zd0, sg0).wait()

        @pl.when(i > 0)
        def _():
            pltpu.make_async_copy(pb0, out_hbm.at[w, j0], so0).wait()

        _compute(zs0, zd0, pb0)
        pltpu.async_copy(pb0, out_hbm.at[w, j0], so0)

        @pl.when(i < CHD // 2 - 1)
        def _():
            pltpu.async_copy(zn_hbm.at[sidx.at[j0 + 2]], zs0, sg0)
            pltpu.async_copy(zn_hbm.at[didx.at[j0 + 2]], zd0, sg0)

        pltpu.make_async_copy(zn_hbm.at[sidx.at[j1]], zs1, sg1).wait()
        pltpu.make_async_copy(zn_hbm.at[didx.at[j1]], zd1, sg1).wait()

        @pl.when(i > 0)
        def _():
            pltpu.make_async_copy(pb1, out_hbm.at[w, j1], so1).wait()

        _compute(zs1, zd1, pb1)
        pltpu.async_copy(pb1, out_hbm.at[w, j1], so1)

        @pl.when(i < CHD // 2 - 1)
        def _():
            pltpu.async_copy(zn_hbm.at[sidx.at[j1 + 2]], zs1, sg1)
            pltpu.async_copy(zn_hbm.at[didx.at[j1 + 2]], zd1, sg1)

        return carry

    lax.fori_loop(0, CHD // 2, body, 0)
    pltpu.make_async_copy(pb0, out_hbm.at[w, CHD - 2], so0).wait()
    pltpu.make_async_copy(pb1, out_hbm.at[w, CHD - 1], so1).wait()


_BR = 2048


def _tc1_body(degp_ref, x_ref, t1_ref, dinv_ref):
    deg = degp_ref[:, 0:1] + degp_ref[:, 1:2] + 1.0
    dinv = lax.rsqrt(jnp.maximum(deg, 1.0))
    t1_ref[...] = jnp.log1p(x_ref[...]) * dinv
    dinv_ref[...] = dinv


def _tc1(degp_t, x_pad):
    return pl.pallas_call(
        _tc1_body,
        grid=(NPAD // _BR,),
        in_specs=[
            pl.BlockSpec((_BR, 2), lambda i: (i, 0)),
            pl.BlockSpec((_BR, D_IN), lambda i: (i, 0)),
        ],
        out_specs=[
            pl.BlockSpec((_BR, D_IN), lambda i: (i, 0)),
            pl.BlockSpec((_BR, 1), lambda i: (i, 0)),
        ],
        out_shape=[
            jax.ShapeDtypeStruct((NPAD, D_IN), jnp.float32),
            jax.ShapeDtypeStruct((NPAD, 1), jnp.float32),
        ],
    )(degp_t, x_pad)


def _tc2_body(p_ref, t1_ref, dinv_ref, w1_ref, wmu_ref, wls_ref,
              t2_ref):
    dv = dinv_ref[...]
    agg1 = (p_ref[0] + p_ref[1] + t1_ref[...]) * dv
    h = jnp.maximum(
        jnp.dot(agg1, w1_ref[...], preferred_element_type=jnp.float32), 0.0)
    hm = jnp.dot(h, wmu_ref[...], preferred_element_type=jnp.float32)
    hs = jnp.dot(h, wls_ref[...], preferred_element_type=jnp.float32)
    t2_ref[...] = jnp.concatenate([hm, hs], axis=1) * dv


def _tc2(parts, t1, dinv, W1, W_mu, W_logstd):
    return pl.pallas_call(
        _tc2_body,
        grid=(NPAD // _BR,),
        in_specs=[
            pl.BlockSpec((2, _BR, D_IN), lambda i: (0, i, 0)),
            pl.BlockSpec((_BR, D_IN), lambda i: (i, 0)),
            pl.BlockSpec((_BR, 1), lambda i: (i, 0)),
            pl.BlockSpec((D_IN, D_HID), lambda i: (0, 0)),
            pl.BlockSpec((D_HID, N_GPS), lambda i: (0, 0)),
            pl.BlockSpec((D_HID, N_GPS), lambda i: (0, 0)),
        ],
        out_specs=pl.BlockSpec((_BR, 2 * N_GPS), lambda i: (i, 0)),
        out_shape=jax.ShapeDtypeStruct((NPAD, 2 * N_GPS), jnp.float32),
    )(parts, t1, dinv, W1, W_mu, W_logstd)


def _tc3_body(q_ref, t2_ref, dinv_ref, wge_ref, mask_ref,
              mu_ref, ls_ref, zn_ref, gep_ref):
    dv = dinv_ref[...]
    m = (q_ref[0] + q_ref[1] + t2_ref[...]) * dv
    mu = m[:, :N_GPS]
    ls = m[:, N_GPS:]
    nrm = jnp.sqrt(jnp.sum(mu * mu, axis=1, keepdims=True))
    zn = mu / (nrm + 1e-8)
    wm = wge_ref[...] * mask_ref[...]
    gl = jnp.dot(mu, wm, preferred_element_type=jnp.float32)
    gmax = jnp.max(gl, axis=1, keepdims=True)
    ge = jnp.exp(gl - gmax)
    gep = ge / jnp.sum(ge, axis=1, keepdims=True)
    mu_ref[...] = mu
    ls_ref[...] = ls
    zn_ref[...] = jnp.concatenate([zn, jnp.zeros_like(zn)], axis=1)
    gep_ref[...] = gep


def _tc3(parts, t2, dinv, W_ge, mask):
    return pl.pallas_call(
        _tc3_body,
        grid=(NPAD // _BR,),
        in_specs=[
            pl.BlockSpec((2, _BR, 2 * N_GPS), lambda i: (0, i, 0)),
            pl.BlockSpec((_BR, 2 * N_GPS), lambda i: (i, 0)),
            pl.BlockSpec((_BR, 1), lambda i: (i, 0)),
            pl.BlockSpec((N_GPS, N_OUT), lambda i: (0, 0)),
            pl.BlockSpec((N_GPS, N_OUT), lambda i: (0, 0)),
        ],
        out_specs=[
            pl.BlockSpec((_BR, N_GPS), lambda i: (i, 0)),
            pl.BlockSpec((_BR, N_GPS), lambda i: (i, 0)),
            pl.BlockSpec((_BR, 2 * N_GPS), lambda i: (i, 0)),
            pl.BlockSpec((_BR, N_OUT), lambda i: (i, 0)),
        ],
        out_shape=[
            jax.ShapeDtypeStruct((NPAD, N_GPS), jnp.float32),
            jax.ShapeDtypeStruct((NPAD, N_GPS), jnp.float32),
            jax.ShapeDtypeStruct((NPAD, 2 * N_GPS), jnp.float32),
            jax.ShapeDtypeStruct((NPAD, N_OUT), jnp.float32),
        ],
    )(parts, t2, dinv, W_ge, mask)


_R16 = NW * CHD * 16
_BRE = 4096


def _tc4_body(p_ref, out_ref):
    p = p_ref[...]
    cols = [jnp.sum(p[:, k * 16:(k + 1) * 16], axis=1) for k in range(8)]
    out_ref[...] = jnp.stack(cols, axis=0)


def _tc4(pfold):
    return pl.pallas_call(
        _tc4_body,
        grid=(_R16 // _BRE,),
        in_specs=[pl.BlockSpec((_BRE, 128), lambda i: (i, 0))],
        out_specs=pl.BlockSpec((8, _BRE), lambda i: (0, i)),
        out_shape=jax.ShapeDtypeStruct((8, _R16), jnp.float32),
    )(pfold)


def kernel(x, edge_index, W1, W_mu, W_logstd, W_ge, mask):
    src = edge_index[0]
    dst = edge_index[1]
    padg = (N + jnp.arange(EPADG - E, dtype=jnp.int32) % (NPAD - N)).astype(
        jnp.int32)
    srcg = jnp.concatenate([src, padg]).reshape(NW, CHG, CKG)
    dstg = jnp.concatenate([dst, padg]).reshape(NW, CHG, CKG)
    padd = (N + jnp.arange(EPADD - E, dtype=jnp.int32) % (NPAD - N)).astype(
        jnp.int32)
    srcd = jnp.concatenate([src, padd]).reshape(NW, CHD, 128)
    dstd = jnp.concatenate([dst, padd]).reshape(NW, CHD, 128)
    x_pad = jnp.pad(x, ((0, NPAD - N), (0, 0)))

    deg_parts = _deg_kernel(dstd)
    t1, dinv = _tc1(deg_parts.T, x_pad)
    parts1 = _agg_kernel(t1, srcg, dstg)
    t2 = _tc2(parts1, t1, dinv, W1, W_mu, W_logstd)
    parts2 = _agg_kernel(t2, srcg, dstg)
    mu_p, ls_p, zn_p, gep_p = _tc3(parts2, t2, dinv, W_ge, mask)
    pfold = _edge_kernel(zn_p, srcd, dstd).reshape(_R16, 128)
    out4 = _tc4(pfold)
    elog = (out4.reshape(8, NW * CHD, 16)
            .transpose(1, 2, 0).reshape(-1)[:E])
    return (elog, gep_p[:N], mu_p[:N], ls_p[:N])
